# Initial kernel scaffold; baseline (speedup 1.0000x reference)
import jax
import jax.numpy as jnp
from jax.experimental import pallas as pl


def kernel(x, edge_index, edge_attr, batch, W1, b1, W2, b2, W3, b3, lin1_W, lin1_b, lin2_W, lin2_b):
    raise NotImplementedError("write your pallas kernel here")


# speedup vs baseline: 1.1063x; 1.1063x over previous
import functools

import jax
import jax.numpy as jnp
from jax import lax
from jax.experimental import pallas as pl
from jax.experimental.pallas import tpu as pltpu
from jax.experimental.pallas import tpu_sc as plsc
from jax.experimental import layout as jlayout


def _lin(arr):
    mtm = tuple(range(arr.ndim))
    return jlayout.with_layout_constraint(
        arr, jlayout.Layout(mtm, tiling=((8,),)))

N = 10000
E = 160000
NUM_GRAPHS = 16

NT = 16
NPAD = 10240
RPT = NPAD // NT
EPAD = 163840
CH = 256
NCH = (EPAD // NT) // CH
CPT = EPAD // NT
ERB = 70
NSP = 10080
RPT_SC = NSP // NT
NEC = RPT_SC // ERB
TRASH = N
TSPREAD = 64

_MESH = dict(core_axis_name="c", subcore_axis_name="s", num_cores=1,
             num_subcores=16)


def _fill_zeros(ref, rows, cols):
    for i in range(rows):
        for j in range(cols // 16):
            ref[i, pl.ds(j * 16, 16)] = jnp.zeros((16,), jnp.float32)


@functools.partial(
    pl.kernel,
    out_type=jax.ShapeDtypeStruct((NPAD,), jnp.float32),
    mesh=plsc.VectorSubcoreMesh(**_MESH),
    scratch_types=[
        pltpu.VMEM((CPT,), jnp.int32),
        pltpu.VMEM((CH,), jnp.float32),
        pltpu.VMEM((RPT,), jnp.float32),
        pltpu.VMEM_SHARED((NPAD,), jnp.float32),
    ],
)
def _deg_kernel(src_hbm, deg_out, src_v, ones_v, zb_v, acc_sh):
    tid = lax.axis_index("s")
    rbase = tid * RPT
    pltpu.sync_copy(src_hbm.at[tid], src_v)
    for k in range(CH // 16):
        ones_v[pl.ds(k * 16, 16)] = jnp.ones((16,), jnp.float32)
    for k in range(RPT // 16):
        zb_v[pl.ds(k * 16, 16)] = jnp.zeros((16,), jnp.float32)
    pltpu.sync_copy(zb_v, acc_sh.at[pl.ds(rbase, RPT)])
    plsc.subcore_barrier()

    def add_body(c, carry):
        pltpu.sync_copy(ones_v, acc_sh.at[src_v.at[pl.ds(c * CH, CH)]],
                        add=True)
        return carry

    lax.fori_loop(0, NCH, add_body, 0)
    plsc.subcore_barrier()
    pltpu.sync_copy(acc_sh.at[pl.ds(rbase, RPT)],
                    deg_out.at[pl.ds(rbase, RPT)])


def _make_layer_kernel(F, NH, NR=4):
    n_in = NH + 4
    out_type = tuple(
        jax.ShapeDtypeStruct((NPAD * F,), jnp.float32) for _ in range(NH))

    @functools.partial(
        pl.kernel,
        out_type=out_type,
        name=f"scprop{F}x{NH}x{NR}",
        mesh=plsc.VectorSubcoreMesh(**_MESH),
        scratch_types=[
            pltpu.VMEM((CPT,), jnp.int32),
            pltpu.VMEM((CPT,), jnp.int32),
            pltpu.VMEM((CH, F), jnp.float32),
            pltpu.VMEM((ERB, F), jnp.float32),
            pltpu.VMEM((ERB, F), jnp.float32),
            pltpu.VMEM((ERB * F,), jnp.float32),
            pltpu.VMEM((ERB * F,), jnp.float32),
            pltpu.VMEM((ERB * F,), jnp.float32),
            pltpu.VMEM((ERB, F), jnp.float32),
            pltpu.VMEM_SHARED((NSP, F), jnp.float32),
            pltpu.VMEM_SHARED((NSP, F), jnp.float32),
            pltpu.SemaphoreType.DMA,
        ],
    )
    def _layer(*refs):
        v0s = refs[:NH]
        zs4, a24, src_hbm, dst_hbm = refs[NH:n_in]
        u_outs = refs[n_in:n_in + NH]
        (src_v, dst_v, rows_v, eacc, e2d, einit, ea2, eout, zb,
         tab_sh, acc_sh, sem) = refs[n_in + NH:]
        tid = lax.axis_index("s")
        rbase = tid * RPT_SC
        pltpu.sync_copy(src_hbm.at[tid], src_v)
        pltpu.sync_copy(dst_hbm.at[tid], dst_v)
        _fill_zeros(zb, ERB, F)

        def zero_body(ch, carry):
            pltpu.sync_copy(zb, acc_sh.at[pl.ds(rbase + ch * ERB, ERB)])
            return carry

        lax.fori_loop(0, NEC, zero_body, 0)

        def stage_from(v0):
            def stage_body(ch, carry):
                row0 = rbase + ch * ERB
                pltpu.sync_copy(v0.at[pl.ds(row0 * F, ERB * F)], eout)
                for i in range(ERB):
                    for j in range(F // 16):
                        e2d[i, pl.ds(j * 16, 16)] = (
                            eout[pl.ds(i * F + j * 16, 16)])
                pltpu.sync_copy(e2d, tab_sh.at[pl.ds(row0, ERB)])
                return carry

            lax.fori_loop(0, NEC, stage_body, 0)

        def round_body(r, carry):
            for h in range(NH):
                @pl.when(r == NR * h)
                def _():
                    stage_from(v0s[h])

            plsc.subcore_barrier()

            def edge_body(c, c2):
                esl = pl.ds(c * CH, CH)
                pltpu.async_copy(tab_sh.at[src_v.at[esl]], rows_v,
                                 sem).wait()
                pltpu.sync_copy(rows_v, acc_sh.at[dst_v.at[esl]], add=True)
                return c2

            lax.fori_loop(0, NCH, edge_body, 0)
            plsc.subcore_barrier()

            def epi_body(ch, c2):
                row0 = rbase + ch * ERB
                pltpu.sync_copy(acc_sh.at[pl.ds(row0, ERB)], eacc)
                pltpu.sync_copy(zs4.at[r, pl.ds(row0 * F, ERB * F)], einit)
                pltpu.sync_copy(a24.at[r, pl.ds(row0 * F, ERB * F)], ea2)
                for i in range(ERB):
                    for j in range(F // 16):
                        sl = pl.ds(i * F + j * 16, 16)
                        slj = pl.ds(j * 16, 16)
                        val = einit[sl] + ea2[sl] * eacc[i, slj]
                        e2d[i, slj] = val
                        eout[sl] = val
                pltpu.sync_copy(e2d, tab_sh.at[pl.ds(row0, ERB)])
                for h in range(NH):
                    @pl.when((r >= NR * h) & (r < NR * (h + 1)))
                    def _():
                        pltpu.sync_copy(
                            eout, u_outs[h].at[pl.ds(row0 * F, ERB * F)])
                pltpu.sync_copy(zb, acc_sh.at[pl.ds(row0, ERB)])
                return c2

            lax.fori_loop(0, NEC, epi_body, 0)
            plsc.subcore_barrier()
            return carry

        lax.fori_loop(0, NR * NH, round_body, 0)

    return _layer


def _prop_layer64(v0, zs, a2f32, srcr, dstfix):
    v0a, v0b = v0[:, 0:32], v0[:, 32:64]
    zs8 = jnp.concatenate(
        [zs[:, :, 0:32].reshape(4, NPAD * 32),
         zs[:, :, 32:64].reshape(4, NPAD * 32)], axis=0)
    a28 = jnp.concatenate([a2f32, a2f32], axis=0)
    ua, ub = _prop64(v0a.reshape(NPAD * 32), v0b.reshape(NPAD * 32),
                     _lin(zs8), _lin(a28), srcr, dstfix)
    return jnp.concatenate(
        [ua.reshape(NPAD, 32), ub.reshape(NPAD, 32)], axis=1)


def _prop_layer(prop_fn, F, v0, zs, a2f, srcr, dstfix):
    (u,) = prop_fn(v0.reshape(NPAD * F), _lin(zs.reshape(4, NPAD * F)),
                   _lin(a2f), srcr, dstfix)
    return u.reshape(NPAD, F)


_prop64 = _make_layer_kernel(32, 2)
_prop16 = _make_layer_kernel(16, 1)
_propdeg = _make_layer_kernel(16, 1, NR=1)


def _proj(h, wc, dis, b, F, leaky):
    nblk = NPAD // 512
    fin = h.shape[1]

    def body(h_ref, w_ref, d_ref, b_ref, v0_ref, zs_ref):
        hh = h_ref[...]
        if leaky:
            hh = jnp.where(hh > 0, hh, 0.01 * hh)
        m = lax.dot_general(hh, w_ref[...], (((1,), (0,)), ((), ())),
                            precision=lax.Precision.HIGHEST,
                            preferred_element_type=jnp.float32)
        d = d_ref[...]
        v0_ref[...] = d * m[:, 0:F]
        zs_ref[0] = d * m[:, F:2 * F]
        zs_ref[1] = d * m[:, 2 * F:3 * F]
        zs_ref[2] = d * m[:, 3 * F:4 * F]
        zs_ref[3] = m[:, 4 * F:5 * F] + b_ref[...]

    return pl.pallas_call(
        body,
        grid=(nblk,),
        in_specs=[
            pl.BlockSpec((512, fin), lambda i: (i, 0)),
            pl.BlockSpec((fin, 5 * F), lambda i: (0, 0)),
            pl.BlockSpec((512, 1), lambda i: (i, 0)),
            pl.BlockSpec((1, F), lambda i: (0, 0)),
        ],
        out_specs=[
            pl.BlockSpec((512, F), lambda i: (i, 0)),
            pl.BlockSpec((4, 512, F), lambda i: (0, i, 0)),
        ],
        out_shape=[
            jax.ShapeDtypeStruct((NPAD, F), jnp.float32),
            jax.ShapeDtypeStruct((4, NPAD, F), jnp.float32),
        ],
    )(h, wc, dis, b)


def _pool_mlp(h, batchf, w1p, b1, w2, b2):
    nblk = NPAD // 512

    def body(h_ref, bat_ref, w1_ref, b1_ref, w2_ref, b2_ref,
             x1_ref, x2_ref, sums):
        i = pl.program_id(0)

        @pl.when(i == 0)
        def _():
            sums[...] = jnp.zeros_like(sums)

        hh = h_ref[...]
        hh = jnp.where(bat_ref[...] < float(NUM_GRAPHS), hh, 0.0)
        hh = jnp.where(hh > 0, hh, 0.01 * hh)
        haug = jnp.concatenate(
            [hh, jnp.ones((512, 1), jnp.float32),
             jnp.zeros((512, 15), jnp.float32)], axis=1)
        g = lax.broadcasted_iota(jnp.int32, (1, NUM_GRAPHS), 1).astype(
            jnp.float32)
        oh = (bat_ref[...] == g).astype(jnp.float32)
        sums[...] += lax.dot_general(oh, haug, (((0,), (0,)), ((), ())),
                                     precision=lax.Precision.HIGHEST,
                                     preferred_element_type=jnp.float32)

        @pl.when(i == nblk - 1)
        def _():
            s = sums[...]
            cnt = jnp.maximum(s[:, 16:17], 1.0)
            pooled = s[:, 0:16] / cnt
            x1 = jnp.maximum(
                lax.dot_general(pooled, w1_ref[...], (((1,), (0,)), ((), ())),
                                precision=lax.Precision.HIGHEST,
                                preferred_element_type=jnp.float32)
                + b1_ref[...], 0.0)
            x1_ref[...] = x1
            x2_ref[...] = lax.dot_general(
                x1, w2_ref[...], (((1,), (0,)), ((), ())),
                precision=lax.Precision.HIGHEST,
                preferred_element_type=jnp.float32) + b2_ref[...]

    return pl.pallas_call(
        body,
        grid=(nblk,),
        in_specs=[
            pl.BlockSpec((512, 16), lambda i: (i, 0)),
            pl.BlockSpec((512, 1), lambda i: (i, 0)),
            pl.BlockSpec((16, 4), lambda i: (0, 0)),
            pl.BlockSpec((1, 4), lambda i: (0, 0)),
            pl.BlockSpec((4, 1), lambda i: (0, 0)),
            pl.BlockSpec((1, 1), lambda i: (0, 0)),
        ],
        out_specs=[
            pl.BlockSpec((NUM_GRAPHS, 4), lambda i: (0, 0)),
            pl.BlockSpec((NUM_GRAPHS, 1), lambda i: (0, 0)),
        ],
        out_shape=[
            jax.ShapeDtypeStruct((NUM_GRAPHS, 4), jnp.float32),
            jax.ShapeDtypeStruct((NUM_GRAPHS, 1), jnp.float32),
        ],
        scratch_shapes=[pltpu.VMEM((NUM_GRAPHS, 32), jnp.float32)],
    )(h, batchf, w1p, b1, w2, b2)


def _cheb_to_monomial(W):
    c4 = 8.0 * W[4]
    c3 = 4.0 * W[3]
    c2 = 2.0 * W[2] - 8.0 * W[4]
    c1 = W[1] - 3.0 * W[3]
    c0 = W[0] - W[2] + W[4]
    return jnp.concatenate([c4, c3, c2, c1, c0], axis=1)


def kernel(x, edge_index, edge_attr, batch, W1, b1, W2, b2, W3, b3,
           lin1_W, lin1_b, lin2_W, lin2_b):
    del edge_attr
    pad_idx = (TRASH + jnp.arange(EPAD - E, dtype=jnp.int32) % TSPREAD)
    srcp = jnp.concatenate([edge_index[0], pad_idx])
    dstp = jnp.concatenate([edge_index[1], pad_idx])
    srcr = _lin(srcp.reshape(NT, CPT))
    dstr = _lin(dstp.reshape(NT, CPT))
    dstfix = _lin(jnp.where(srcp == dstp, TRASH, dstp).reshape(NT, CPT))

    deg = _deg_kernel(srcr)[:N]
    pos = deg > 0
    dis = jnp.where(pos, lax.rsqrt(jnp.maximum(deg, 1.0)), 0.0)
    a_mid = jnp.where(pos, -1.0 / jnp.maximum(deg, 1.0), 0.0)
    a_last = -dis

    padn = NPAD - N
    dis_pad = jnp.pad(dis, (0, padn))[:, None]
    a_stack = jnp.stack([a_mid, a_mid, a_mid, a_last])
    a_stack = jnp.pad(a_stack, ((0, 0), (0, padn)))
    a2_32 = jnp.broadcast_to(a_stack[:, :, None],
                             (4, NPAD, 32)).reshape(4, NPAD * 32)
    a2_16 = jnp.broadcast_to(a_stack[:, :, None],
                             (4, NPAD, 16)).reshape(4, NPAD * 16)

    x_pad = jnp.pad(x, ((0, padn), (0, 0)))

    sfl = srcr.reshape(-1)
    dfl = dstfix.reshape(-1)
    amid_p = jnp.pad(a_mid, (0, padn))
    alast_p = jnp.pad(a_last, (0, padn))

    def _prop_rounds(v0_, zs_):
        u = v0_
        for r in range(4):
            s = jax.ops.segment_sum(u[sfl], dfl,
                                    num_segments=NPAD + TSPREAD)[:NPAD]
            aa = amid_p if r < 3 else alast_p
            u = zs_[r] + aa[:, None] * s
        return u

    v0, zs = _proj(x_pad, _cheb_to_monomial(W1), dis_pad, b1[None, :], 64,
                   leaky=False)
    u1 = _prop_rounds(v0, zs)
    v0, zs = _proj(u1, _cheb_to_monomial(W2), dis_pad, b2[None, :], 16,
                   leaky=True)
    u2 = _prop_rounds(v0, zs)
    W3p = jnp.pad(W3, ((0, 0), (0, 0), (0, 8)))
    b3p = jnp.pad(b3, (0, 8))
    v0, zs = _proj(u2, _cheb_to_monomial(W3p), dis_pad, b3p[None, :], 16,
                   leaky=True)
    u3 = _prop_rounds(v0, zs)

    batchf = jnp.pad(batch.astype(jnp.float32), (0, padn),
                     constant_values=float(NUM_GRAPHS))[:, None]
    w1p = jnp.pad(lin1_W, ((0, 8), (0, 0)))
    x1, x2 = _pool_mlp(u3, batchf, w1p, lin1_b[None, :], lin2_W,
                       lin2_b[None, :])
    return (x1, x2)


# speedup vs baseline: 1.1065x; 1.0002x over previous
import functools

import jax
import jax.numpy as jnp
from jax import lax
from jax.experimental import pallas as pl
from jax.experimental.pallas import tpu as pltpu
from jax.experimental.pallas import tpu_sc as plsc
from jax.experimental import layout as jlayout


def _lin(arr):
    mtm = tuple(range(arr.ndim))
    return jlayout.with_layout_constraint(
        arr, jlayout.Layout(mtm, tiling=((8,),)))

N = 10000
E = 160000
NUM_GRAPHS = 16

NT = 16
NPAD = 10240
RPT = NPAD // NT
EPAD = 163840
CH = 256
NCH = (EPAD // NT) // CH
CPT = EPAD // NT
TRASH = N
TSPREAD = 64

_MESH = dict(core_axis_name="c", subcore_axis_name="s", num_cores=1,
             num_subcores=16)


@functools.partial(
    pl.kernel,
    out_type=jax.ShapeDtypeStruct((NPAD,), jnp.float32),
    mesh=plsc.VectorSubcoreMesh(**_MESH),
    scratch_types=[
        pltpu.VMEM((CPT,), jnp.int32),
        pltpu.VMEM((CH,), jnp.float32),
        pltpu.VMEM((RPT,), jnp.float32),
        pltpu.VMEM_SHARED((NPAD,), jnp.float32),
    ],
)
def _deg_kernel(src_hbm, deg_out, src_v, ones_v, zb_v, acc_sh):
    tid = lax.axis_index("s")
    rbase = tid * RPT
    pltpu.sync_copy(src_hbm.at[tid], src_v)
    for k in range(CH // 16):
        ones_v[pl.ds(k * 16, 16)] = jnp.ones((16,), jnp.float32)
    for k in range(RPT // 16):
        zb_v[pl.ds(k * 16, 16)] = jnp.zeros((16,), jnp.float32)
    pltpu.sync_copy(zb_v, acc_sh.at[pl.ds(rbase, RPT)])
    plsc.subcore_barrier()

    def add_body(c, carry):
        pltpu.sync_copy(ones_v, acc_sh.at[src_v.at[pl.ds(c * CH, CH)]],
                        add=True)
        return carry

    lax.fori_loop(0, NCH, add_body, 0)
    plsc.subcore_barrier()
    pltpu.sync_copy(acc_sh.at[pl.ds(rbase, RPT)],
                    deg_out.at[pl.ds(rbase, RPT)])


def _proj(h, wc, dis, b, F, leaky):
    nblk = NPAD // 512
    fin = h.shape[1]

    def body(h_ref, w_ref, d_ref, b_ref, v0_ref, zs_ref):
        hh = h_ref[...]
        if leaky:
            hh = jnp.where(hh > 0, hh, 0.01 * hh)
        m = lax.dot_general(hh, w_ref[...], (((1,), (0,)), ((), ())),
                            precision=lax.Precision.HIGHEST,
                            preferred_element_type=jnp.float32)
        d = d_ref[...]
        v0_ref[...] = d * m[:, 0:F]
        zs_ref[0] = d * m[:, F:2 * F]
        zs_ref[1] = d * m[:, 2 * F:3 * F]
        zs_ref[2] = d * m[:, 3 * F:4 * F]
        zs_ref[3] = m[:, 4 * F:5 * F] + b_ref[...]

    return pl.pallas_call(
        body,
        grid=(nblk,),
        in_specs=[
            pl.BlockSpec((512, fin), lambda i: (i, 0)),
            pl.BlockSpec((fin, 5 * F), lambda i: (0, 0)),
            pl.BlockSpec((512, 1), lambda i: (i, 0)),
            pl.BlockSpec((1, F), lambda i: (0, 0)),
        ],
        out_specs=[
            pl.BlockSpec((512, F), lambda i: (i, 0)),
            pl.BlockSpec((4, 512, F), lambda i: (0, i, 0)),
        ],
        out_shape=[
            jax.ShapeDtypeStruct((NPAD, F), jnp.float32),
            jax.ShapeDtypeStruct((4, NPAD, F), jnp.float32),
        ],
    )(h, wc, dis, b)


def _pool_mlp(h, batchf, w1p, b1, w2, b2):
    nblk = NPAD // 512

    def body(h_ref, bat_ref, w1_ref, b1_ref, w2_ref, b2_ref,
             x1_ref, x2_ref, sums):
        i = pl.program_id(0)

        @pl.when(i == 0)
        def _():
            sums[...] = jnp.zeros_like(sums)

        hh = h_ref[...]
        hh = jnp.where(bat_ref[...] < float(NUM_GRAPHS), hh, 0.0)
        hh = jnp.where(hh > 0, hh, 0.01 * hh)
        haug = jnp.concatenate(
            [hh, jnp.ones((512, 1), jnp.float32),
             jnp.zeros((512, 15), jnp.float32)], axis=1)
        g = lax.broadcasted_iota(jnp.int32, (1, NUM_GRAPHS), 1).astype(
            jnp.float32)
        oh = (bat_ref[...] == g).astype(jnp.float32)
        sums[...] += lax.dot_general(oh, haug, (((0,), (0,)), ((), ())),
                                     precision=lax.Precision.HIGHEST,
                                     preferred_element_type=jnp.float32)

        @pl.when(i == nblk - 1)
        def _():
            s = sums[...]
            cnt = jnp.maximum(s[:, 16:17], 1.0)
            pooled = s[:, 0:16] / cnt
            x1 = jnp.maximum(
                lax.dot_general(pooled, w1_ref[...], (((1,), (0,)), ((), ())),
                                precision=lax.Precision.HIGHEST,
                                preferred_element_type=jnp.float32)
                + b1_ref[...], 0.0)
            x1_ref[...] = x1
            x2_ref[...] = lax.dot_general(
                x1, w2_ref[...], (((1,), (0,)), ((), ())),
                precision=lax.Precision.HIGHEST,
                preferred_element_type=jnp.float32) + b2_ref[...]

    return pl.pallas_call(
        body,
        grid=(nblk,),
        in_specs=[
            pl.BlockSpec((512, 16), lambda i: (i, 0)),
            pl.BlockSpec((512, 1), lambda i: (i, 0)),
            pl.BlockSpec((16, 4), lambda i: (0, 0)),
            pl.BlockSpec((1, 4), lambda i: (0, 0)),
            pl.BlockSpec((4, 1), lambda i: (0, 0)),
            pl.BlockSpec((1, 1), lambda i: (0, 0)),
        ],
        out_specs=[
            pl.BlockSpec((NUM_GRAPHS, 4), lambda i: (0, 0)),
            pl.BlockSpec((NUM_GRAPHS, 1), lambda i: (0, 0)),
        ],
        out_shape=[
            jax.ShapeDtypeStruct((NUM_GRAPHS, 4), jnp.float32),
            jax.ShapeDtypeStruct((NUM_GRAPHS, 1), jnp.float32),
        ],
        scratch_shapes=[pltpu.VMEM((NUM_GRAPHS, 32), jnp.float32)],
    )(h, batchf, w1p, b1, w2, b2)


def _cheb_to_monomial(W):
    c4 = 8.0 * W[4]
    c3 = 4.0 * W[3]
    c2 = 2.0 * W[2] - 8.0 * W[4]
    c1 = W[1] - 3.0 * W[3]
    c0 = W[0] - W[2] + W[4]
    return jnp.concatenate([c4, c3, c2, c1, c0], axis=1)


def kernel(x, edge_index, edge_attr, batch, W1, b1, W2, b2, W3, b3,
           lin1_W, lin1_b, lin2_W, lin2_b):
    del edge_attr
    pad_idx = (TRASH + jnp.arange(EPAD - E, dtype=jnp.int32) % TSPREAD)
    srcp = jnp.concatenate([edge_index[0], pad_idx])
    dstp = jnp.concatenate([edge_index[1], pad_idx])
    srcr = _lin(srcp.reshape(NT, CPT))
    dstfix = jnp.where(srcp == dstp, TRASH, dstp)

    deg = _deg_kernel(srcr)[:N]
    pos = deg > 0
    dis = jnp.where(pos, lax.rsqrt(jnp.maximum(deg, 1.0)), 0.0)
    a_mid = jnp.where(pos, -1.0 / jnp.maximum(deg, 1.0), 0.0)
    a_last = -dis

    padn = NPAD - N
    dis_pad = jnp.pad(dis, (0, padn))[:, None]
    a_stack = jnp.stack([a_mid, a_mid, a_mid, a_last])
    a_stack = jnp.pad(a_stack, ((0, 0), (0, padn)))
    a2_32 = jnp.broadcast_to(a_stack[:, :, None],
                             (4, NPAD, 32)).reshape(4, NPAD * 32)
    a2_16 = jnp.broadcast_to(a_stack[:, :, None],
                             (4, NPAD, 16)).reshape(4, NPAD * 16)

    x_pad = jnp.pad(x, ((0, padn), (0, 0)))

    sfl = srcp
    dfl = dstfix
    amid_p = jnp.pad(a_mid, (0, padn))
    alast_p = jnp.pad(a_last, (0, padn))

    def _prop_rounds(v0_, zs_):
        u = v0_
        for r in range(4):
            s = jax.ops.segment_sum(u[sfl], dfl,
                                    num_segments=NPAD + TSPREAD)[:NPAD]
            aa = amid_p if r < 3 else alast_p
            u = zs_[r] + aa[:, None] * s
        return u

    v0, zs = _proj(x_pad, _cheb_to_monomial(W1), dis_pad, b1[None, :], 64,
                   leaky=False)
    u1 = _prop_rounds(v0, zs)
    v0, zs = _proj(u1, _cheb_to_monomial(W2), dis_pad, b2[None, :], 16,
                   leaky=True)
    u2 = _prop_rounds(v0, zs)
    W3p = jnp.pad(W3, ((0, 0), (0, 0), (0, 8)))
    b3p = jnp.pad(b3, (0, 8))
    v0, zs = _proj(u2, _cheb_to_monomial(W3p), dis_pad, b3p[None, :], 16,
                   leaky=True)
    u3 = _prop_rounds(v0, zs)

    batchf = jnp.pad(batch.astype(jnp.float32), (0, padn),
                     constant_values=float(NUM_GRAPHS))[:, None]
    w1p = jnp.pad(lin1_W, ((0, 8), (0, 0)))
    x1, x2 = _pool_mlp(u3, batchf, w1p, lin1_b[None, :], lin2_W,
                       lin2_b[None, :])
    return (x1, x2)
